# initial kernel scaffold (unmeasured)
import jax
import jax.numpy as jnp
from jax import lax
from jax.experimental import pallas as pl
from jax.experimental.pallas import tpu as pltpu

N_DEV = 4
STRIP = 512


def kernel(A, B):
    m_per, k = A.shape
    _, n = B.shape
    n_strips = m_per // STRIP

    A16 = A.astype(jnp.bfloat16)
    B16 = B.astype(jnp.bfloat16)

    def body(a_ref, b_ref, out_ref, comm_ref, c_ref, send_sems, recv_sems, copy_sem):
        my = lax.axis_index("i")
        left = (my - 1) % N_DEV
        right = (my + 1) % N_DEV

        barrier_sem = pltpu.get_barrier_semaphore()
        for nbr in (left, right):
            pl.semaphore_signal(
                barrier_sem, inc=1,
                device_id=(nbr,), device_id_type=pl.DeviceIdType.MESH,
            )
        pl.semaphore_wait(barrier_sem, 2)

        cp_in = pltpu.make_async_copy(a_ref, comm_ref.at[0], copy_sem)
        cp_in.start()
        cp_in.wait()

        for h in range(N_DEV):
            slot = h % 2
            if h < N_DEV - 1:
                rdma = pltpu.make_async_remote_copy(
                    src_ref=comm_ref.at[slot],
                    dst_ref=comm_ref.at[1 - slot],
                    send_sem=send_sems.at[slot],
                    recv_sem=recv_sems.at[1 - slot],
                    device_id=(right,),
                    device_id_type=pl.DeviceIdType.MESH,
                )
                rdma.start()

            origin = (my - h) % N_DEV
            for s in range(n_strips):
                a_strip = comm_ref[slot, s * STRIP:(s + 1) * STRIP, :]
                c_ref[...] = jnp.dot(
                    a_strip, b_ref[...], preferred_element_type=jnp.float32
                )
                cp_out = pltpu.make_async_copy(
                    c_ref,
                    out_ref.at[pl.ds(origin * m_per + s * STRIP, STRIP), :],
                    copy_sem,
                )
                cp_out.start()
                cp_out.wait()

            if h < N_DEV - 1:
                rdma.wait()

    return pl.pallas_call(
        body,
        out_shape=jax.ShapeDtypeStruct((N_DEV * m_per, n), jnp.float32),
        in_specs=[
            pl.BlockSpec(memory_space=pltpu.ANY),
            pl.BlockSpec(memory_space=pltpu.VMEM),
        ],
        out_specs=pl.BlockSpec(memory_space=pltpu.ANY),
        scratch_shapes=[
            pltpu.VMEM((2, m_per, k), jnp.bfloat16),
            pltpu.VMEM((STRIP, n), jnp.float32),
            pltpu.SemaphoreType.DMA((2,)),
            pltpu.SemaphoreType.DMA((2,)),
            pltpu.SemaphoreType.DMA,
        ],
        compiler_params=pltpu.CompilerParams(collective_id=0),
    )(A16, B16)


# baseline (device time: 860072 ns/iter reference)
import jax
import jax.numpy as jnp
from jax import lax
from jax.experimental import pallas as pl
from jax.experimental.pallas import tpu as pltpu

N_DEV = 4
STRIP = 512


def kernel(A, B):
    m_per, k = A.shape
    _, n = B.shape
    n_strips = m_per // STRIP

    A16 = A.astype(jnp.bfloat16)
    B16 = B.astype(jnp.bfloat16)

    def body(a_ref, b_ref, out_ref, comm_ref, c_ref, send_sems, recv_sems, copy_sem):
        my = lax.axis_index("i")
        left = (my - 1) % N_DEV
        right = (my + 1) % N_DEV

        barrier_sem = pltpu.get_barrier_semaphore()
        for nbr in (left, right):
            pl.semaphore_signal(
                barrier_sem, inc=1,
                device_id=(nbr,), device_id_type=pl.DeviceIdType.MESH,
            )
        pl.semaphore_wait(barrier_sem, 2)

        cp_in = pltpu.make_async_copy(a_ref, comm_ref.at[0], copy_sem)
        cp_in.start()
        cp_in.wait()

        for h in range(N_DEV):
            slot = h % 2
            if h < N_DEV - 1:
                rdma = pltpu.make_async_remote_copy(
                    src_ref=comm_ref.at[slot],
                    dst_ref=comm_ref.at[1 - slot],
                    send_sem=send_sems.at[slot],
                    recv_sem=recv_sems.at[1 - slot],
                    device_id=(right,),
                    device_id_type=pl.DeviceIdType.MESH,
                )
                rdma.start()

            origin = (my - h) % N_DEV
            row0 = origin * m_per

            def strip_step(s, _, slot=slot):
                a_strip = comm_ref[slot, pl.ds(s * STRIP, STRIP), :]
                c_ref[...] = jnp.dot(
                    a_strip, b_ref[...], preferred_element_type=jnp.float32
                )
                cp_out = pltpu.make_async_copy(
                    c_ref,
                    out_ref.at[pl.ds(row0 + s * STRIP, STRIP), :],
                    copy_sem,
                )
                cp_out.start()
                cp_out.wait()
                return _

            lax.fori_loop(0, n_strips, strip_step, None)

            if h < N_DEV - 1:
                rdma.wait()

    return pl.pallas_call(
        body,
        out_shape=jax.ShapeDtypeStruct((N_DEV * m_per, n), jnp.float32),
        in_specs=[
            pl.BlockSpec(memory_space=pl.ANY),
            pl.BlockSpec(memory_space=pltpu.VMEM),
        ],
        out_specs=pl.BlockSpec(memory_space=pl.ANY),
        scratch_shapes=[
            pltpu.VMEM((2, m_per, k), jnp.bfloat16),
            pltpu.VMEM((STRIP, n), jnp.float32),
            pltpu.SemaphoreType.DMA((2,)),
            pltpu.SemaphoreType.DMA((2,)),
            pltpu.SemaphoreType.DMA,
        ],
        compiler_params=pltpu.CompilerParams(
            collective_id=0, vmem_limit_bytes=64 * 1024 * 1024
        ),
    )(A16, B16)


# device time: 568244 ns/iter; 1.5136x vs baseline; 1.5136x over previous
import jax
import jax.numpy as jnp
from jax import lax
from jax.experimental import pallas as pl
from jax.experimental.pallas import tpu as pltpu

N_DEV = 4
STRIP = 256


def kernel(A, B):
    m_per, k = A.shape
    _, n = B.shape
    m_half = m_per // 2
    n_strips = m_half // STRIP
    n_pairs = n_strips // 2

    A16 = A.astype(jnp.bfloat16)
    B16 = B.astype(jnp.bfloat16)

    def body(a_ref, b_ref, out_ref,
             cw_ref, ccw_ref, c0_ref, c1_ref,
             in_sems, out_sems,
             send_cw, recv_cw, send_ccw, recv_ccw):
        my = lax.axis_index("i")
        left = (my - 1) % N_DEV
        right = (my + 1) % N_DEV

        cp_top = pltpu.make_async_copy(
            a_ref.at[pl.ds(0, m_half), :], cw_ref.at[0], in_sems.at[0]
        )
        cp_bot = pltpu.make_async_copy(
            a_ref.at[pl.ds(m_half, m_half), :], ccw_ref.at[0], in_sems.at[1]
        )
        cp_top.start()
        cp_bot.start()

        barrier_sem = pltpu.get_barrier_semaphore()
        for nbr in (left, right):
            pl.semaphore_signal(
                barrier_sem, inc=1,
                device_id=(nbr,), device_id_type=pl.DeviceIdType.MESH,
            )
        pl.semaphore_wait(barrier_sem, 2)
        cp_top.wait()
        cp_bot.wait()

        def out_desc(c_ref, sem, row):
            return pltpu.make_async_copy(
                c_ref, out_ref.at[pl.ds(row, STRIP), :], sem
            )

        def compute_half(comm_ref, slot, row_base):
            def pair_step(p, _):
                s0 = 2 * p

                @pl.when(p >= 1)
                def _():
                    out_desc(c0_ref, out_sems.at[0], row_base).wait()

                c0_ref[...] = jnp.dot(
                    comm_ref[slot, pl.ds(s0 * STRIP, STRIP), :],
                    b_ref[...], preferred_element_type=jnp.float32,
                )
                out_desc(c0_ref, out_sems.at[0], row_base + s0 * STRIP).start()

                @pl.when(p >= 1)
                def _():
                    out_desc(c1_ref, out_sems.at[1], row_base).wait()

                c1_ref[...] = jnp.dot(
                    comm_ref[slot, pl.ds((s0 + 1) * STRIP, STRIP), :],
                    b_ref[...], preferred_element_type=jnp.float32,
                )
                out_desc(c1_ref, out_sems.at[1], row_base + (s0 + 1) * STRIP).start()
                return _

            lax.fori_loop(0, n_pairs, pair_step, None)
            out_desc(c0_ref, out_sems.at[0], row_base).wait()
            out_desc(c1_ref, out_sems.at[1], row_base).wait()

        for h in range(N_DEV):
            slot = h % 2
            if h < N_DEV - 1:
                rdma_cw = pltpu.make_async_remote_copy(
                    src_ref=cw_ref.at[slot],
                    dst_ref=cw_ref.at[1 - slot],
                    send_sem=send_cw.at[slot],
                    recv_sem=recv_cw.at[1 - slot],
                    device_id=(right,),
                    device_id_type=pl.DeviceIdType.MESH,
                )
                rdma_ccw = pltpu.make_async_remote_copy(
                    src_ref=ccw_ref.at[slot],
                    dst_ref=ccw_ref.at[1 - slot],
                    send_sem=send_ccw.at[slot],
                    recv_sem=recv_ccw.at[1 - slot],
                    device_id=(left,),
                    device_id_type=pl.DeviceIdType.MESH,
                )
                rdma_cw.start()
                rdma_ccw.start()

            origin_cw = (my - h) % N_DEV
            origin_ccw = (my + h) % N_DEV
            compute_half(cw_ref, slot, origin_cw * m_per)
            compute_half(ccw_ref, slot, origin_ccw * m_per + m_half)

            if h < N_DEV - 1:
                rdma_cw.wait()
                rdma_ccw.wait()

    return pl.pallas_call(
        body,
        out_shape=jax.ShapeDtypeStruct((N_DEV * m_per, n), jnp.float32),
        in_specs=[
            pl.BlockSpec(memory_space=pl.ANY),
            pl.BlockSpec(memory_space=pltpu.VMEM),
        ],
        out_specs=pl.BlockSpec(memory_space=pl.ANY),
        scratch_shapes=[
            pltpu.VMEM((2, m_per // 2, k), jnp.bfloat16),
            pltpu.VMEM((2, m_per // 2, k), jnp.bfloat16),
            pltpu.VMEM((STRIP, n), jnp.float32),
            pltpu.VMEM((STRIP, n), jnp.float32),
            pltpu.SemaphoreType.DMA((2,)),
            pltpu.SemaphoreType.DMA((2,)),
            pltpu.SemaphoreType.DMA((2,)),
            pltpu.SemaphoreType.DMA((2,)),
            pltpu.SemaphoreType.DMA((2,)),
            pltpu.SemaphoreType.DMA((2,)),
        ],
        compiler_params=pltpu.CompilerParams(
            collective_id=0, vmem_limit_bytes=64 * 1024 * 1024
        ),
    )(A16, B16)
